# baseline (speedup 1.0000x reference)
import functools

import jax
import jax.numpy as jnp
from jax import lax
from jax.experimental import pallas as pl
from jax.experimental.pallas import tpu as pltpu
from jax.experimental.pallas import tpu_sc as plsc

N = 16384
D = 64
_PORT_V = 65536
_PROTO_V = 256
_INFO = plsc.get_sparse_core_info()
_NC, _NS = _INFO.num_cores, _INFO.num_subcores
_NW = _NC * _NS
_BPW = N // _NW
_CHUNK = 128
_NBUF = 4

_MESH = plsc.VectorSubcoreMesh(core_axis_name="c", subcore_axis_name="s")


@functools.partial(
    pl.kernel,
    mesh=_MESH,
    compiler_params=pltpu.CompilerParams(needs_layout_passes=False),
    out_type=jax.ShapeDtypeStruct((N, 2 * D), jnp.float32),
    scratch_types=[
        [pltpu.VMEM((_CHUNK,), jnp.int32) for _ in range(_BPW // _CHUNK)],
        [pltpu.VMEM((_CHUNK, 2 * D), jnp.float32) for _ in range(_NBUF)],
        [pltpu.SemaphoreType.DMA for _ in range(_NBUF)],
        [pltpu.SemaphoreType.DMA for _ in range(_NBUF)],
    ],
)
def _sc_gather(table128, idx_hbm, out_hbm, idx_bufs, pair_bufs, gsems, wsems):
    wid = lax.axis_index("s") * _NC + lax.axis_index("c")
    base = wid * _BPW
    nchunks = _BPW // _CHUNK
    assert nchunks == _NBUF
    for c in range(nchunks):
        pltpu.sync_copy(idx_hbm.at[pl.ds(base + c * _CHUNK, _CHUNK)],
                        idx_bufs[c])
    gathers = [
        pltpu.async_copy(table128.at[idx_bufs[c]], pair_bufs[c], gsems[c])
        for c in range(nchunks)
    ]
    writes = []
    for c, g in enumerate(gathers):
        g.wait()
        writes.append(pltpu.async_copy(
            pair_bufs[c], out_hbm.at[pl.ds(base + c * _CHUNK, _CHUNK)],
            wsems[c]))
    for w in writes:
        w.wait()


_TBLK = 2048


def _tp_body(x_ref, o_ref):
    o_ref[:, 0:D] = x_ref[...].T


def _row_major_padded(table_t, vocab):
    blk = min(_TBLK, vocab)
    return pl.pallas_call(
        _tp_body,
        grid=(vocab // blk,),
        in_specs=[pl.BlockSpec((D, blk), lambda i: (0, i))],
        out_specs=pl.BlockSpec((blk, 2 * D), lambda i: (i, 0)),
        out_shape=jax.ShapeDtypeStruct((vocab, 2 * D), jnp.float32),
    )(table_t)


def _ip_body(w_ref, x_ref, b_ref, o_ref):
    o_ref[...] = (
        lax.dot_general(w_ref[...], x_ref[...], (((0,), (0,)), ((), ())),
                        preferred_element_type=jnp.float32)
        + b_ref[...]
    )


_IP_BLK = 4096


def _ip_linear(ip_features_t, W_ip, b_ip):
    return pl.pallas_call(
        _ip_body,
        grid=(N // _IP_BLK,),
        in_specs=[
            pl.BlockSpec((32, D), lambda i: (0, 0)),
            pl.BlockSpec((32, _IP_BLK), lambda i: (0, i)),
            pl.BlockSpec((D, 1), lambda i: (0, 0)),
        ],
        out_specs=pl.BlockSpec((D, _IP_BLK), lambda i: (0, i)),
        out_shape=jax.ShapeDtypeStruct((D, N), jnp.float32),
    )(W_ip, ip_features_t, b_ip.reshape(D, 1))


def kernel(ip_features, port_indices, proto_indices, W_ip, b_ip,
           port_table, proto_table):
    proto128 = _row_major_padded(proto_table.T, _PROTO_V)
    xq128 = _sc_gather(proto128, proto_indices.astype(jnp.int32))
    ip_t, _ = lax.optimization_barrier((ip_features.T, proto128))
    x_ip_t = _ip_linear(ip_t, W_ip, b_ip)
    port_t, _ = lax.optimization_barrier((port_table.T, x_ip_t))
    port128 = _row_major_padded(port_t, _PORT_V)
    xp128 = _sc_gather(port128, port_indices.astype(jnp.int32))
    return (x_ip_t.T, xp128[:, :D], xq128[:, :D])


# speedup vs baseline: 1.0759x; 1.0759x over previous
import functools

import jax
import jax.numpy as jnp
from jax import lax
from jax.experimental import pallas as pl
from jax.experimental.pallas import tpu as pltpu
from jax.experimental.pallas import tpu_sc as plsc

N = 16384
D = 64
_PORT_V = 65536
_PROTO_V = 256
_INFO = plsc.get_sparse_core_info()
_NC, _NS = _INFO.num_cores, _INFO.num_subcores
_NW = _NC * _NS
_BPW = N // _NW
_CHUNK = 128
_NBUF = 4

_MESH = plsc.VectorSubcoreMesh(core_axis_name="c", subcore_axis_name="s")


@functools.partial(
    pl.kernel,
    mesh=_MESH,
    compiler_params=pltpu.CompilerParams(needs_layout_passes=False),
    out_type=jax.ShapeDtypeStruct((N, 2 * D), jnp.float32),
    scratch_types=[
        [pltpu.VMEM((_CHUNK,), jnp.int32) for _ in range(_BPW // _CHUNK)],
        [pltpu.VMEM((_CHUNK, 2 * D), jnp.float32) for _ in range(_NBUF)],
        [pltpu.SemaphoreType.DMA for _ in range(_NBUF)],
        [pltpu.SemaphoreType.DMA for _ in range(_NBUF)],
    ],
)
def _sc_gather(table128, idx_hbm, out_hbm, idx_bufs, pair_bufs, gsems, wsems):
    wid = lax.axis_index("s") * _NC + lax.axis_index("c")
    base = wid * _BPW
    nchunks = _BPW // _CHUNK
    assert nchunks == _NBUF
    for c in range(nchunks):
        pltpu.sync_copy(idx_hbm.at[pl.ds(base + c * _CHUNK, _CHUNK)],
                        idx_bufs[c])
    gathers = [
        pltpu.async_copy(table128.at[idx_bufs[c]], pair_bufs[c], gsems[c])
        for c in range(nchunks)
    ]
    writes = []
    for c, g in enumerate(gathers):
        g.wait()
        writes.append(pltpu.async_copy(
            pair_bufs[c], out_hbm.at[pl.ds(base + c * _CHUNK, _CHUNK)],
            wsems[c]))
    for w in writes:
        w.wait()


_TBLK = 2048


def _tp_body(x_ref, o_ref):
    o_ref[:, 0:D] = x_ref[...].T


def _row_major_padded(table_t, vocab):
    blk = min(_TBLK, vocab)
    return pl.pallas_call(
        _tp_body,
        grid=(vocab // blk,),
        in_specs=[pl.BlockSpec((D, blk), lambda i: (0, i))],
        out_specs=pl.BlockSpec((blk, 2 * D), lambda i: (i, 0)),
        out_shape=jax.ShapeDtypeStruct((vocab, 2 * D), jnp.float32),
    )(table_t)


def _ip_body(w_ref, x_ref, b_ref, o_ref):
    o_ref[...] = (
        lax.dot_general(w_ref[...], x_ref[...], (((0,), (0,)), ((), ())),
                        preferred_element_type=jnp.float32)
        + b_ref[...]
    )


_IP_BLK = 4096


def _ip_linear(ip_features_t, W_ip, b_ip):
    return pl.pallas_call(
        _ip_body,
        grid=(N // _IP_BLK,),
        in_specs=[
            pl.BlockSpec((32, D), lambda i: (0, 0)),
            pl.BlockSpec((32, _IP_BLK), lambda i: (0, i)),
            pl.BlockSpec((D, 1), lambda i: (0, 0)),
        ],
        out_specs=pl.BlockSpec((D, _IP_BLK), lambda i: (0, i)),
        out_shape=jax.ShapeDtypeStruct((D, N), jnp.float32),
    )(W_ip, ip_features_t, b_ip.reshape(D, 1))


def kernel(ip_features, port_indices, proto_indices, W_ip, b_ip,
           port_table, proto_table):
    x_ip_t = _ip_linear(ip_features.T, W_ip, b_ip)
    proto128 = _row_major_padded(proto_table.T, _PROTO_V)
    xq128 = _sc_gather(proto128, proto_indices.astype(jnp.int32))
    port128 = _row_major_padded(port_table.T, _PORT_V)
    xp128 = _sc_gather(port128, port_indices.astype(jnp.int32))
    return (x_ip_t.T, xp128[:, :D], xq128[:, :D])


# speedup vs baseline: 1.2593x; 1.1705x over previous
import functools

import jax
import jax.numpy as jnp
from jax import lax
from jax.experimental import pallas as pl
from jax.experimental.pallas import tpu as pltpu
from jax.experimental.pallas import tpu_sc as plsc

N = 16384
D = 64
_PORT_V = 65536
_PROTO_V = 256
_INFO = plsc.get_sparse_core_info()
_NC, _NS = _INFO.num_cores, _INFO.num_subcores
_NW = _NC * _NS
_BPW = N // _NW
_CHUNK = 128
_NBUF = 4

_MESH = plsc.VectorSubcoreMesh(core_axis_name="c", subcore_axis_name="s")


@functools.partial(
    pl.kernel,
    mesh=_MESH,
    compiler_params=pltpu.CompilerParams(needs_layout_passes=False),
    out_type=jax.ShapeDtypeStruct((N, 2 * D), jnp.float32),
    scratch_types=[
        [pltpu.VMEM((_CHUNK,), jnp.int32) for _ in range(_BPW // _CHUNK)],
        [pltpu.VMEM((_CHUNK, 2 * D), jnp.float32) for _ in range(_NBUF)],
        [pltpu.SemaphoreType.DMA for _ in range(_NBUF)],
        [pltpu.SemaphoreType.DMA for _ in range(_NBUF)],
    ],
)
def _sc_gather(table128, idx_hbm, out_hbm, idx_bufs, pair_bufs, gsems, wsems):
    wid = lax.axis_index("s") * _NC + lax.axis_index("c")
    base = wid * _BPW
    nchunks = _BPW // _CHUNK
    assert nchunks == _NBUF
    for c in range(nchunks):
        pltpu.sync_copy(idx_hbm.at[pl.ds(base + c * _CHUNK, _CHUNK)],
                        idx_bufs[c])
    gathers = [
        pltpu.async_copy(table128.at[idx_bufs[c]], pair_bufs[c], gsems[c])
        for c in range(nchunks)
    ]
    writes = []
    for c, g in enumerate(gathers):
        g.wait()
        writes.append(pltpu.async_copy(
            pair_bufs[c], out_hbm.at[pl.ds(base + c * _CHUNK, _CHUNK)],
            wsems[c]))
    for w in writes:
        w.wait()


_TBLK = 8192


def _tp_body(x_ref, o_ref):
    o_ref[:, 0:D] = x_ref[...].T


def _row_major_padded(table_t, vocab):
    blk = min(_TBLK, vocab)
    return pl.pallas_call(
        _tp_body,
        grid=(vocab // blk,),
        in_specs=[pl.BlockSpec((D, blk), lambda i: (0, i))],
        out_specs=pl.BlockSpec((blk, 2 * D), lambda i: (i, 0)),
        out_shape=jax.ShapeDtypeStruct((vocab, 2 * D), jnp.float32),
    )(table_t)


def _ip_body(w_ref, x_ref, b_ref, o_ref):
    o_ref[...] = (
        lax.dot_general(w_ref[...], x_ref[...], (((0,), (0,)), ((), ())),
                        preferred_element_type=jnp.float32)
        + b_ref[...]
    )


_IP_BLK = 4096


def _ip_linear(ip_features_t, W_ip, b_ip):
    return pl.pallas_call(
        _ip_body,
        grid=(N // _IP_BLK,),
        in_specs=[
            pl.BlockSpec((32, D), lambda i: (0, 0)),
            pl.BlockSpec((32, _IP_BLK), lambda i: (0, i)),
            pl.BlockSpec((D, 1), lambda i: (0, 0)),
        ],
        out_specs=pl.BlockSpec((D, _IP_BLK), lambda i: (0, i)),
        out_shape=jax.ShapeDtypeStruct((D, N), jnp.float32),
    )(W_ip, ip_features_t, b_ip.reshape(D, 1))


def kernel(ip_features, port_indices, proto_indices, W_ip, b_ip,
           port_table, proto_table):
    x_ip_t = _ip_linear(ip_features.T, W_ip, b_ip)
    proto128 = _row_major_padded(proto_table.T, _PROTO_V)
    xq128 = _sc_gather(proto128, proto_indices.astype(jnp.int32))
    port128 = _row_major_padded(port_table.T, _PORT_V)
    xp128 = _sc_gather(port128, port_indices.astype(jnp.int32))
    return (x_ip_t.T, xp128[:, :D], xq128[:, :D])


# speedup vs baseline: 1.2902x; 1.0245x over previous
import functools

import jax
import jax.numpy as jnp
from jax import lax
from jax.experimental import pallas as pl
from jax.experimental.pallas import tpu as pltpu
from jax.experimental.pallas import tpu_sc as plsc

N = 16384
D = 64
_PORT_V = 65536
_PROTO_V = 256
_INFO = plsc.get_sparse_core_info()
_NC, _NS = _INFO.num_cores, _INFO.num_subcores
_NW = _NC * _NS
_BPW = N // _NW
_CHUNK = 128
_NBUF = 4

_MESH = plsc.VectorSubcoreMesh(core_axis_name="c", subcore_axis_name="s")


@functools.partial(
    pl.kernel,
    mesh=_MESH,
    compiler_params=pltpu.CompilerParams(needs_layout_passes=False),
    out_type=jax.ShapeDtypeStruct((N, 2 * D), jnp.float32),
    scratch_types=[
        [pltpu.VMEM((_CHUNK,), jnp.int32) for _ in range(_BPW // _CHUNK)],
        [pltpu.VMEM((_CHUNK, 2 * D), jnp.float32) for _ in range(_NBUF)],
        [pltpu.SemaphoreType.DMA for _ in range(_NBUF)],
        [pltpu.SemaphoreType.DMA for _ in range(_NBUF)],
    ],
)
def _sc_gather(table128, idx_hbm, out_hbm, idx_bufs, pair_bufs, gsems, wsems):
    wid = lax.axis_index("s") * _NC + lax.axis_index("c")
    base = wid * _BPW
    nchunks = _BPW // _CHUNK
    assert nchunks == _NBUF
    for c in range(nchunks):
        pltpu.sync_copy(idx_hbm.at[pl.ds(base + c * _CHUNK, _CHUNK)],
                        idx_bufs[c])
    gathers = [
        pltpu.async_copy(table128.at[idx_bufs[c]], pair_bufs[c], gsems[c])
        for c in range(nchunks)
    ]
    writes = []
    for c, g in enumerate(gathers):
        g.wait()
        writes.append(pltpu.async_copy(
            pair_bufs[c], out_hbm.at[pl.ds(base + c * _CHUNK, _CHUNK)],
            wsems[c]))
    for w in writes:
        w.wait()


_TBLK = 16384


def _tp_body(x_ref, o_ref):
    o_ref[:, 0:D] = x_ref[...].T


def _row_major_padded(table_t, vocab):
    blk = min(_TBLK, vocab)
    return pl.pallas_call(
        _tp_body,
        grid=(vocab // blk,),
        in_specs=[pl.BlockSpec((D, blk), lambda i: (0, i))],
        out_specs=pl.BlockSpec((blk, 2 * D), lambda i: (i, 0)),
        out_shape=jax.ShapeDtypeStruct((vocab, 2 * D), jnp.float32),
    )(table_t)


def _ip_body(w_ref, x_ref, b_ref, o_ref):
    o_ref[...] = (
        lax.dot_general(w_ref[...], x_ref[...], (((0,), (0,)), ((), ())),
                        preferred_element_type=jnp.float32)
        + b_ref[...]
    )


_IP_BLK = 4096


def _ip_linear(ip_features_t, W_ip, b_ip):
    return pl.pallas_call(
        _ip_body,
        grid=(N // _IP_BLK,),
        in_specs=[
            pl.BlockSpec((32, D), lambda i: (0, 0)),
            pl.BlockSpec((32, _IP_BLK), lambda i: (0, i)),
            pl.BlockSpec((D, 1), lambda i: (0, 0)),
        ],
        out_specs=pl.BlockSpec((D, _IP_BLK), lambda i: (0, i)),
        out_shape=jax.ShapeDtypeStruct((D, N), jnp.float32),
    )(W_ip, ip_features_t, b_ip.reshape(D, 1))


def kernel(ip_features, port_indices, proto_indices, W_ip, b_ip,
           port_table, proto_table):
    x_ip_t = _ip_linear(ip_features.T, W_ip, b_ip)
    proto128 = _row_major_padded(proto_table.T, _PROTO_V)
    xq128 = _sc_gather(proto128, proto_indices.astype(jnp.int32))
    port128 = _row_major_padded(port_table.T, _PORT_V)
    xp128 = _sc_gather(port128, port_indices.astype(jnp.int32))
    return (x_ip_t.T, xp128[:, :D], xq128[:, :D])
